# baseline (speedup 1.0000x reference)
import jax
import jax.numpy as jnp
from jax.experimental import pallas as pl

_B = 256
_H = 1024
_NS = 4


def _matvec_body(xT_ref, *refs):
    A_refs = refs[:_NS]
    Q_refs = refs[_NS:2 * _NS]
    yblk_ref = refs[2 * _NS]
    Ax_ref, Qx_ref, ATy_ref = refs[2 * _NS + 1:]
    i = pl.program_id(0)
    xT = xT_ref[...]
    yb = yblk_ref[...]
    dn_row = (((1,), (1,)), ((), ()))
    dn_col = (((1,), (0,)), ((), ()))
    f32 = jnp.float32
    ax = None
    qx = None
    for s in range(_NS):
        xs = xT[:, s * _H:(s + 1) * _H]
        a = jax.lax.dot_general(xs, A_refs[s][...], dn_row,
                                preferred_element_type=f32)
        q = jax.lax.dot_general(xs, Q_refs[s][...], dn_row,
                                preferred_element_type=f32)
        ax = a if ax is None else ax + a
        qx = q if qx is None else qx + q
    Ax_ref[...] = ax
    Qx_ref[...] = qx
    for s in range(_NS):
        cs = jax.lax.dot_general(yb, A_refs[s][...], dn_col,
                                 preferred_element_type=f32)
        sl = slice(s * _H, (s + 1) * _H)

        @pl.when(i == 0)
        def _init(cs=cs, sl=sl):
            ATy_ref[:, sl] = cs

        @pl.when(i > 0)
        def _acc(cs=cs, sl=sl):
            ATy_ref[:, sl] = ATy_ref[:, sl] + cs


def _finalize_body(Ax_ref, Qx_ref, ATy_ref, b_ref, c_ref, x_ref, y_ref,
                   Iy_ref, il_ref, iu_ref, l_ref, u_ref,
                   res_ref, t1_ref, t2_ref, t3_ref):
    relu = lambda v: jnp.maximum(v, 0.0)
    Ax = Ax_ref[...]
    Qx = Qx_ref[...]
    ATy = ATy_ref[...]
    b = b_ref[...]
    c = c_ref[...]
    x = x_ref[...]
    y = y_ref[...]
    Iy = Iy_ref[...]
    il = il_ref[...]
    iu = iu_ref[...]
    l = l_ref[...]
    u = u_ref[...]
    cons = Ax - b
    cons = cons + relu(-cons) * Iy
    var = relu(l - x) * il + relu(x - u) * iu
    part2 = jnp.maximum(jnp.max(jnp.abs(var)), jnp.max(jnp.abs(cons)))
    t1 = part2 / (1.0 + jnp.max(jnp.abs(b)))
    quad = jnp.sum(x * Qx)
    lin = jnp.sum(c * x)
    vio = jnp.sum(b * y)
    pg_g = c - ATy + Qx
    RC = relu(pg_g) * il - relu(-pg_g) * iu
    tm = jnp.where(RC > 0, l, u)
    rc = jnp.sum(RC * tm)
    top_g = jnp.abs(quad + lin - vio - rc)
    bot_g = 1.0 + jnp.maximum(jnp.abs(vio - 0.5 * quad),
                              jnp.abs(0.5 * quad + lin))
    t3 = top_g / bot_g
    pg = c + ATy + Qx
    RCV = pg - relu(pg) * il - relu(-pg) * iu
    DR = relu(-y) * Iy
    t2 = jnp.maximum(jnp.max(jnp.abs(RCV)), jnp.max(jnp.abs(DR))) / \
        (1.0 + jnp.max(jnp.abs(c)))
    res_ref[...] = jnp.reshape(t1 + t2 + t3, (1, 1))
    t1_ref[...] = jnp.reshape(t1, (1, 1))
    t2_ref[...] = jnp.reshape(t2, (1, 1))
    t3_ref[...] = jnp.reshape(t3, (1, 1))


def kernel(Q, A, AT, b, c, x, y, Iy, il, iu, l, u):
    del AT
    m, n = A.shape
    nb = m // _B
    xT = x.reshape(1, n)
    yT = y.reshape(1, m)

    Ax, Qx, ATy = pl.pallas_call(
        _matvec_body,
        grid=(nb,),
        in_specs=(
            [pl.BlockSpec((1, n), lambda i: (0, 0))]
            + [pl.BlockSpec((_B, _H), lambda i, s=s: (i, s))
               for s in range(_NS)]
            + [pl.BlockSpec((_B, _H), lambda i, s=s: (i, s))
               for s in range(_NS)]
            + [pl.BlockSpec((1, _B), lambda i: (0, i))]
        ),
        out_specs=[
            pl.BlockSpec((1, _B), lambda i: (0, i)),
            pl.BlockSpec((1, _B), lambda i: (0, i)),
            pl.BlockSpec((1, n), lambda i: (0, 0)),
        ],
        out_shape=[
            jax.ShapeDtypeStruct((1, m), jnp.float32),
            jax.ShapeDtypeStruct((1, n), jnp.float32),
            jax.ShapeDtypeStruct((1, n), jnp.float32),
        ],
    )(xT, *([A] * _NS), *([Q] * _NS), yT)

    shp = (32, n // 32)
    sd = jax.ShapeDtypeStruct((1, 1), jnp.float32)
    res, t1, t2, t3 = pl.pallas_call(
        _finalize_body,
        out_shape=[sd, sd, sd, sd],
    )(Ax.reshape(shp), Qx.reshape(shp), ATy.reshape(shp),
      b.reshape(shp), c.reshape(shp), x.reshape(shp), y.reshape(shp),
      Iy.reshape(shp), il.reshape(shp), iu.reshape(shp),
      l.reshape(shp), u.reshape(shp))
    return (res, t1.reshape(()), t2.reshape(()), t3)


# speedup vs baseline: 1.0584x; 1.0584x over previous
import jax
import jax.numpy as jnp
from jax.experimental import pallas as pl

_B = 512
_H = 2048
_NS = 2


def _matvec_body(xT_ref, *refs):
    A_refs = refs[:_NS]
    Q_refs = refs[_NS:2 * _NS]
    yblk_ref = refs[2 * _NS]
    Ax_ref, Qx_ref, ATy_ref = refs[2 * _NS + 1:]
    i = pl.program_id(0)
    xT = xT_ref[...]
    yb = yblk_ref[...]
    dn_row = (((1,), (1,)), ((), ()))
    dn_col = (((1,), (0,)), ((), ()))
    f32 = jnp.float32
    ax = None
    qx = None
    for s in range(_NS):
        xs = xT[:, s * _H:(s + 1) * _H]
        a = jax.lax.dot_general(xs, A_refs[s][...], dn_row,
                                preferred_element_type=f32)
        q = jax.lax.dot_general(xs, Q_refs[s][...], dn_row,
                                preferred_element_type=f32)
        ax = a if ax is None else ax + a
        qx = q if qx is None else qx + q
    Ax_ref[...] = ax
    Qx_ref[...] = qx
    for s in range(_NS):
        cs = jax.lax.dot_general(yb, A_refs[s][...], dn_col,
                                 preferred_element_type=f32)
        sl = slice(s * _H, (s + 1) * _H)

        @pl.when(i == 0)
        def _init(cs=cs, sl=sl):
            ATy_ref[:, sl] = cs

        @pl.when(i > 0)
        def _acc(cs=cs, sl=sl):
            ATy_ref[:, sl] = ATy_ref[:, sl] + cs


def _finalize_body(Ax_ref, Qx_ref, ATy_ref, b_ref, c_ref, x_ref, y_ref,
                   Iy_ref, il_ref, iu_ref, l_ref, u_ref,
                   res_ref, t1_ref, t2_ref, t3_ref):
    relu = lambda v: jnp.maximum(v, 0.0)
    Ax = Ax_ref[...]
    Qx = Qx_ref[...]
    ATy = ATy_ref[...]
    b = b_ref[...]
    c = c_ref[...]
    x = x_ref[...]
    y = y_ref[...]
    Iy = Iy_ref[...]
    il = il_ref[...]
    iu = iu_ref[...]
    l = l_ref[...]
    u = u_ref[...]
    cons = Ax - b
    cons = cons + relu(-cons) * Iy
    var = relu(l - x) * il + relu(x - u) * iu
    part2 = jnp.maximum(jnp.max(jnp.abs(var)), jnp.max(jnp.abs(cons)))
    t1 = part2 / (1.0 + jnp.max(jnp.abs(b)))
    quad = jnp.sum(x * Qx)
    lin = jnp.sum(c * x)
    vio = jnp.sum(b * y)
    pg_g = c - ATy + Qx
    RC = relu(pg_g) * il - relu(-pg_g) * iu
    tm = jnp.where(RC > 0, l, u)
    rc = jnp.sum(RC * tm)
    top_g = jnp.abs(quad + lin - vio - rc)
    bot_g = 1.0 + jnp.maximum(jnp.abs(vio - 0.5 * quad),
                              jnp.abs(0.5 * quad + lin))
    t3 = top_g / bot_g
    pg = c + ATy + Qx
    RCV = pg - relu(pg) * il - relu(-pg) * iu
    DR = relu(-y) * Iy
    t2 = jnp.maximum(jnp.max(jnp.abs(RCV)), jnp.max(jnp.abs(DR))) / \
        (1.0 + jnp.max(jnp.abs(c)))
    res_ref[...] = jnp.reshape(t1 + t2 + t3, (1, 1))
    t1_ref[...] = jnp.reshape(t1, (1, 1))
    t2_ref[...] = jnp.reshape(t2, (1, 1))
    t3_ref[...] = jnp.reshape(t3, (1, 1))


def kernel(Q, A, AT, b, c, x, y, Iy, il, iu, l, u):
    del AT
    m, n = A.shape
    nb = m // _B
    xT = x.reshape(1, n)
    yT = y.reshape(1, m)

    Ax, Qx, ATy = pl.pallas_call(
        _matvec_body,
        grid=(nb,),
        in_specs=(
            [pl.BlockSpec((1, n), lambda i: (0, 0))]
            + [pl.BlockSpec((_B, _H), lambda i, s=s: (i, s))
               for s in range(_NS)]
            + [pl.BlockSpec((_B, _H), lambda i, s=s: (i, s))
               for s in range(_NS)]
            + [pl.BlockSpec((1, _B), lambda i: (0, i))]
        ),
        out_specs=[
            pl.BlockSpec((1, _B), lambda i: (0, i)),
            pl.BlockSpec((1, _B), lambda i: (0, i)),
            pl.BlockSpec((1, n), lambda i: (0, 0)),
        ],
        out_shape=[
            jax.ShapeDtypeStruct((1, m), jnp.float32),
            jax.ShapeDtypeStruct((1, n), jnp.float32),
            jax.ShapeDtypeStruct((1, n), jnp.float32),
        ],
    )(xT, *([A] * _NS), *([Q] * _NS), yT)

    shp = (32, n // 32)
    sd = jax.ShapeDtypeStruct((1, 1), jnp.float32)
    res, t1, t2, t3 = pl.pallas_call(
        _finalize_body,
        out_shape=[sd, sd, sd, sd],
    )(Ax.reshape(shp), Qx.reshape(shp), ATy.reshape(shp),
      b.reshape(shp), c.reshape(shp), x.reshape(shp), y.reshape(shp),
      Iy.reshape(shp), il.reshape(shp), iu.reshape(shp),
      l.reshape(shp), u.reshape(shp))
    return (res, t1.reshape(()), t2.reshape(()), t3)


# speedup vs baseline: 1.0672x; 1.0083x over previous
import jax
import jax.numpy as jnp
from jax.experimental import pallas as pl

_B = 256


def _matvec_body(xT_ref, At_ref, Ab_ref, Qt_ref, Qb_ref, yt_ref, yb_ref,
                 Axt_ref, Axb_ref, Qxt_ref, Qxb_ref, ATy_ref):
    i = pl.program_id(0)
    xT = xT_ref[...]
    yt = yt_ref[...]
    yb = yb_ref[...]
    dn_row = (((1,), (1,)), ((), ()))
    dn_col = (((1,), (0,)), ((), ()))
    f32 = jnp.float32
    At = At_ref[...]
    Ab = Ab_ref[...]
    Axt_ref[...] = jax.lax.dot_general(xT, At, dn_row,
                                       preferred_element_type=f32)
    Axb_ref[...] = jax.lax.dot_general(xT, Ab, dn_row,
                                       preferred_element_type=f32)
    Qxt_ref[...] = jax.lax.dot_general(xT, Qt_ref[...], dn_row,
                                       preferred_element_type=f32)
    Qxb_ref[...] = jax.lax.dot_general(xT, Qb_ref[...], dn_row,
                                       preferred_element_type=f32)
    contrib = (jax.lax.dot_general(yt, At, dn_col,
                                   preferred_element_type=f32)
               + jax.lax.dot_general(yb, Ab, dn_col,
                                     preferred_element_type=f32))

    @pl.when(i == 0)
    def _init():
        ATy_ref[...] = contrib

    @pl.when(i > 0)
    def _acc():
        ATy_ref[...] = ATy_ref[...] + contrib


def _finalize_body(Axt_ref, Axb_ref, Qxt_ref, Qxb_ref, ATy_ref,
                   b_ref, c_ref, x_ref, y_ref,
                   Iy_ref, il_ref, iu_ref, l_ref, u_ref,
                   res_ref, t1_ref, t2_ref, t3_ref):
    relu = lambda v: jnp.maximum(v, 0.0)
    Ax = jnp.concatenate([Axt_ref[...], Axb_ref[...]], axis=0)
    Qx = jnp.concatenate([Qxt_ref[...], Qxb_ref[...]], axis=0)
    ATy = ATy_ref[...]
    b = b_ref[...]
    c = c_ref[...]
    x = x_ref[...]
    y = y_ref[...]
    Iy = Iy_ref[...]
    il = il_ref[...]
    iu = iu_ref[...]
    l = l_ref[...]
    u = u_ref[...]
    cons = Ax - b
    cons = cons + relu(-cons) * Iy
    var = relu(l - x) * il + relu(x - u) * iu
    part2 = jnp.maximum(jnp.max(jnp.abs(var)), jnp.max(jnp.abs(cons)))
    t1 = part2 / (1.0 + jnp.max(jnp.abs(b)))
    quad = jnp.sum(x * Qx)
    lin = jnp.sum(c * x)
    vio = jnp.sum(b * y)
    pg_g = c - ATy + Qx
    RC = relu(pg_g) * il - relu(-pg_g) * iu
    tm = jnp.where(RC > 0, l, u)
    rc = jnp.sum(RC * tm)
    top_g = jnp.abs(quad + lin - vio - rc)
    bot_g = 1.0 + jnp.maximum(jnp.abs(vio - 0.5 * quad),
                              jnp.abs(0.5 * quad + lin))
    t3 = top_g / bot_g
    pg = c + ATy + Qx
    RCV = pg - relu(pg) * il - relu(-pg) * iu
    DR = relu(-y) * Iy
    t2 = jnp.maximum(jnp.max(jnp.abs(RCV)), jnp.max(jnp.abs(DR))) / \
        (1.0 + jnp.max(jnp.abs(c)))
    res_ref[...] = jnp.reshape(t1 + t2 + t3, (1, 1))
    t1_ref[...] = jnp.reshape(t1, (1, 1))
    t2_ref[...] = jnp.reshape(t2, (1, 1))
    t3_ref[...] = jnp.reshape(t3, (1, 1))


def kernel(Q, A, AT, b, c, x, y, Iy, il, iu, l, u):
    del AT
    m, n = A.shape
    h = m // 2
    nb = h // _B
    xT = x.reshape(1, n)
    yT = y.reshape(1, m)
    hb = jax.ShapeDtypeStruct((1, h), jnp.float32)

    Axt, Axb, Qxt, Qxb, ATy = pl.pallas_call(
        _matvec_body,
        grid=(nb,),
        in_specs=[
            pl.BlockSpec((1, n), lambda i: (0, 0)),
            pl.BlockSpec((_B, n), lambda i: (i, 0)),
            pl.BlockSpec((_B, n), lambda i: (i + nb, 0)),
            pl.BlockSpec((_B, n), lambda i: (i, 0)),
            pl.BlockSpec((_B, n), lambda i: (i + nb, 0)),
            pl.BlockSpec((1, _B), lambda i: (0, i)),
            pl.BlockSpec((1, _B), lambda i: (0, i + nb)),
        ],
        out_specs=[
            pl.BlockSpec((1, _B), lambda i: (0, i)),
            pl.BlockSpec((1, _B), lambda i: (0, i)),
            pl.BlockSpec((1, _B), lambda i: (0, i)),
            pl.BlockSpec((1, _B), lambda i: (0, i)),
            pl.BlockSpec((1, n), lambda i: (0, 0)),
        ],
        out_shape=[hb, hb, hb, hb,
                   jax.ShapeDtypeStruct((1, n), jnp.float32)],
    )(xT, A, A, Q, Q, yT, yT)

    hs = (16, n // 32)
    shp = (32, n // 32)
    sd = jax.ShapeDtypeStruct((1, 1), jnp.float32)
    res, t1, t2, t3 = pl.pallas_call(
        _finalize_body,
        out_shape=[sd, sd, sd, sd],
    )(Axt.reshape(hs), Axb.reshape(hs), Qxt.reshape(hs), Qxb.reshape(hs),
      ATy.reshape(shp),
      b.reshape(shp), c.reshape(shp), x.reshape(shp), y.reshape(shp),
      Iy.reshape(shp), il.reshape(shp), iu.reshape(shp),
      l.reshape(shp), u.reshape(shp))
    return (res, t1.reshape(()), t2.reshape(()), t3)


# speedup vs baseline: 1.0938x; 1.0249x over previous
import jax
import jax.numpy as jnp
from jax.experimental import pallas as pl

_B = 256
_H = 2048


def _matvec_body(xT_ref, A0_ref, A1_ref, Q0_ref, Q1_ref, yblk_ref,
                 Ax_ref, Qx_ref, ATy_ref):
    i = pl.program_id(0)
    xT = xT_ref[...]
    yb = yblk_ref[...]
    x0 = xT[:, :_H]
    x1 = xT[:, _H:]
    dn_row = (((1,), (1,)), ((), ()))
    dn_col = (((1,), (0,)), ((), ()))
    A0 = A0_ref[...]
    A1 = A1_ref[...]
    Q0 = Q0_ref[...]
    Q1 = Q1_ref[...]
    f32 = jnp.float32
    Ax_ref[...] = (
        jax.lax.dot_general(x0, A0, dn_row, preferred_element_type=f32)
        + jax.lax.dot_general(x1, A1, dn_row, preferred_element_type=f32))
    Qx_ref[...] = (
        jax.lax.dot_general(x0, Q0, dn_row, preferred_element_type=f32)
        + jax.lax.dot_general(x1, Q1, dn_row, preferred_element_type=f32))
    c0 = jax.lax.dot_general(yb, A0, dn_col, preferred_element_type=f32)
    c1 = jax.lax.dot_general(yb, A1, dn_col, preferred_element_type=f32)

    @pl.when(i == 0)
    def _init():
        ATy_ref[:, :_H] = c0
        ATy_ref[:, _H:] = c1

    @pl.when(i > 0)
    def _acc():
        ATy_ref[:, :_H] = ATy_ref[:, :_H] + c0
        ATy_ref[:, _H:] = ATy_ref[:, _H:] + c1


def _finalize_body(Ax_ref, Qx_ref, ATy_ref, b_ref, c_ref, x_ref, y_ref,
                   Iy_ref, il_ref, iu_ref, l_ref, u_ref,
                   res_ref, t1_ref, t2_ref, t3_ref):
    relu = lambda v: jnp.maximum(v, 0.0)
    Ax = Ax_ref[...]
    Qx = Qx_ref[...]
    ATy = ATy_ref[...]
    b = b_ref[...]
    c = c_ref[...]
    x = x_ref[...]
    y = y_ref[...]
    Iy = Iy_ref[...]
    il = il_ref[...]
    iu = iu_ref[...]
    l = l_ref[...]
    u = u_ref[...]
    cons = Ax - b
    cons = cons + relu(-cons) * Iy
    var = relu(l - x) * il + relu(x - u) * iu
    part2 = jnp.maximum(jnp.max(jnp.abs(var)), jnp.max(jnp.abs(cons)))
    t1 = part2 / (1.0 + jnp.max(jnp.abs(b)))
    quad = jnp.sum(x * Qx)
    lin = jnp.sum(c * x)
    vio = jnp.sum(b * y)
    pg_g = c - ATy + Qx
    RC = relu(pg_g) * il - relu(-pg_g) * iu
    tm = jnp.where(RC > 0, l, u)
    rc = jnp.sum(RC * tm)
    top_g = jnp.abs(quad + lin - vio - rc)
    bot_g = 1.0 + jnp.maximum(jnp.abs(vio - 0.5 * quad),
                              jnp.abs(0.5 * quad + lin))
    t3 = top_g / bot_g
    pg = c + ATy + Qx
    RCV = pg - relu(pg) * il - relu(-pg) * iu
    DR = relu(-y) * Iy
    t2 = jnp.maximum(jnp.max(jnp.abs(RCV)), jnp.max(jnp.abs(DR))) / \
        (1.0 + jnp.max(jnp.abs(c)))
    res_ref[...] = jnp.reshape(t1 + t2 + t3, (1, 1))
    t1_ref[...] = jnp.reshape(t1, (1, 1))
    t2_ref[...] = jnp.reshape(t2, (1, 1))
    t3_ref[...] = jnp.reshape(t3, (1, 1))


def kernel(Q, A, AT, b, c, x, y, Iy, il, iu, l, u):
    del AT
    m, n = A.shape
    nb = m // _B
    xT = x.reshape(1, n)
    yT = y.reshape(1, m)

    Ax, Qx, ATy = pl.pallas_call(
        _matvec_body,
        grid=(nb,),
        in_specs=[
            pl.BlockSpec((1, n), lambda i: (0, 0)),
            pl.BlockSpec((_B, _H), lambda i: (i, 0)),
            pl.BlockSpec((_B, _H), lambda i: (i, 1)),
            pl.BlockSpec((_B, _H), lambda i: (i, 0)),
            pl.BlockSpec((_B, _H), lambda i: (i, 1)),
            pl.BlockSpec((1, _B), lambda i: (0, i)),
        ],
        out_specs=[
            pl.BlockSpec((1, _B), lambda i: (0, i)),
            pl.BlockSpec((1, _B), lambda i: (0, i)),
            pl.BlockSpec((1, n), lambda i: (0, 0)),
        ],
        out_shape=[
            jax.ShapeDtypeStruct((1, m), jnp.float32),
            jax.ShapeDtypeStruct((1, n), jnp.float32),
            jax.ShapeDtypeStruct((1, n), jnp.float32),
        ],
    )(xT, A, A, Q, Q, yT)

    shp = (32, n // 32)
    sd = jax.ShapeDtypeStruct((1, 1), jnp.float32)
    res, t1, t2, t3 = pl.pallas_call(
        _finalize_body,
        out_shape=[sd, sd, sd, sd],
    )(Ax.reshape(shp), Qx.reshape(shp), ATy.reshape(shp),
      b.reshape(shp), c.reshape(shp), x.reshape(shp), y.reshape(shp),
      Iy.reshape(shp), il.reshape(shp), iu.reshape(shp),
      l.reshape(shp), u.reshape(shp))
    return (res, t1.reshape(()), t2.reshape(()), t3)


# speedup vs baseline: 1.1671x; 1.0670x over previous
import jax
import jax.numpy as jnp
from jax.experimental import pallas as pl
from jax.experimental.pallas import tpu as pltpu

_B = 256
_H = 2048


def _body(xT_ref, A0_ref, A1_ref, Q0_ref, Q1_ref, yblk_ref,
          b_ref, c_ref, x3_ref, y3_ref, Iy_ref, il_ref, iu_ref,
          l_ref, u_ref,
          res_ref, t1_ref, t2_ref, t3_ref,
          Axs_ref, Qxs_ref, ATy_ref):
    i = pl.program_id(0)
    nb = pl.num_programs(0)
    xT = xT_ref[...]
    yb = yblk_ref[...]
    x0 = xT[:, :_H]
    x1 = xT[:, _H:]
    dn_row = (((1,), (1,)), ((), ()))
    dn_col = (((1,), (0,)), ((), ()))
    A0 = A0_ref[...]
    A1 = A1_ref[...]
    Q0 = Q0_ref[...]
    Q1 = Q1_ref[...]
    f32 = jnp.float32
    Axs_ref[i] = (
        jax.lax.dot_general(x0, A0, dn_row, preferred_element_type=f32)
        + jax.lax.dot_general(x1, A1, dn_row, preferred_element_type=f32))
    Qxs_ref[i] = (
        jax.lax.dot_general(x0, Q0, dn_row, preferred_element_type=f32)
        + jax.lax.dot_general(x1, Q1, dn_row, preferred_element_type=f32))
    c0 = jax.lax.dot_general(yb, A0, dn_col, preferred_element_type=f32)
    c1 = jax.lax.dot_general(yb, A1, dn_col, preferred_element_type=f32)

    @pl.when(i == 0)
    def _init():
        ATy_ref[:, :_H] = c0
        ATy_ref[:, _H:] = c1

    @pl.when(i > 0)
    def _acc():
        ATy_ref[:, :_H] = ATy_ref[:, :_H] + c0
        ATy_ref[:, _H:] = ATy_ref[:, _H:] + c1

    @pl.when(i == nb - 1)
    def _finalize():
        relu = lambda v: jnp.maximum(v, 0.0)
        shp = Axs_ref.shape
        Ax = Axs_ref[...]
        Qx = Qxs_ref[...]
        ATy = ATy_ref[...].reshape(shp)
        b = b_ref[...]
        c = c_ref[...]
        x = x3_ref[...]
        y = y3_ref[...]
        Iy = Iy_ref[...]
        il = il_ref[...]
        iu = iu_ref[...]
        l = l_ref[...]
        u = u_ref[...]
        cons = Ax - b
        cons = cons + relu(-cons) * Iy
        var = relu(l - x) * il + relu(x - u) * iu
        part2 = jnp.maximum(jnp.max(jnp.abs(var)), jnp.max(jnp.abs(cons)))
        t1 = part2 / (1.0 + jnp.max(jnp.abs(b)))
        quad = jnp.sum(x * Qx)
        lin = jnp.sum(c * x)
        vio = jnp.sum(b * y)
        pg_g = c - ATy + Qx
        RC = relu(pg_g) * il - relu(-pg_g) * iu
        tm = jnp.where(RC > 0, l, u)
        rc = jnp.sum(RC * tm)
        top_g = jnp.abs(quad + lin - vio - rc)
        bot_g = 1.0 + jnp.maximum(jnp.abs(vio - 0.5 * quad),
                                  jnp.abs(0.5 * quad + lin))
        t3 = top_g / bot_g
        pg = c + ATy + Qx
        RCV = pg - relu(pg) * il - relu(-pg) * iu
        DR = relu(-y) * Iy
        t2 = jnp.maximum(jnp.max(jnp.abs(RCV)), jnp.max(jnp.abs(DR))) / \
            (1.0 + jnp.max(jnp.abs(c)))
        res_ref[...] = jnp.reshape(t1 + t2 + t3, (1, 1))
        t1_ref[...] = jnp.reshape(t1, (1, 1))
        t2_ref[...] = jnp.reshape(t2, (1, 1))
        t3_ref[...] = jnp.reshape(t3, (1, 1))


def kernel(Q, A, AT, b, c, x, y, Iy, il, iu, l, u):
    del AT
    m, n = A.shape
    nb = m // _B
    xT = x.reshape(1, n)
    yT = y.reshape(1, m)
    shp3 = (nb, 1, _B)
    vec_spec = pl.BlockSpec(shp3, lambda i: (0, 0, 0))
    sd = jax.ShapeDtypeStruct((1, 1), jnp.float32)
    sc_spec = pl.BlockSpec((1, 1), lambda i: (0, 0))

    res, t1, t2, t3 = pl.pallas_call(
        _body,
        grid=(nb,),
        in_specs=[
            pl.BlockSpec((1, n), lambda i: (0, 0)),
            pl.BlockSpec((_B, _H), lambda i: (i, 0)),
            pl.BlockSpec((_B, _H), lambda i: (i, 1)),
            pl.BlockSpec((_B, _H), lambda i: (i, 0)),
            pl.BlockSpec((_B, _H), lambda i: (i, 1)),
            pl.BlockSpec((1, _B), lambda i: (0, i)),
        ] + [vec_spec] * 9,
        out_specs=[sc_spec, sc_spec, sc_spec, sc_spec],
        out_shape=[sd, sd, sd, sd],
        scratch_shapes=[
            pltpu.VMEM(shp3, jnp.float32),
            pltpu.VMEM(shp3, jnp.float32),
            pltpu.VMEM((1, n), jnp.float32),
        ],
    )(xT, A, A, Q, Q, yT,
      b.reshape(shp3), c.reshape(shp3), x.reshape(shp3), y.reshape(shp3),
      Iy.reshape(shp3), il.reshape(shp3), iu.reshape(shp3),
      l.reshape(shp3), u.reshape(shp3))
    return (res, t1.reshape(()), t2.reshape(()), t3)
